# Initial kernel scaffold; baseline (speedup 1.0000x reference)
import jax
import jax.numpy as jnp
from jax.experimental import pallas as pl


def kernel(edge_index, edge_type, queries, var_ids, q_graphs, emb, W_rgcn, W_self, var_emb, W_q_msg, W_q_self):
    raise NotImplementedError("write your pallas kernel here")


# speedup vs baseline: 20.6434x; 20.6434x over previous
import functools

import jax
import jax.numpy as jnp
from jax import lax
from jax.experimental import pallas as pl
from jax.experimental.pallas import tpu as pltpu
from jax.experimental.pallas import tpu_sc as plsc

N = 10000
R = 8
D = 128
L = 3
E = 320000
Q = 1024
A = 3
V = 4
G = 256

NW = 32
CH = 128
CPT = 79
EPAD = NW * CPT * CH
NPAD = N + 112
RPT = NPAD // 16

BN = 2000
NB = N // BN

@functools.lru_cache(maxsize=None)
def _sc_mesh():
    return plsc.VectorSubcoreMesh(core_axis_name="c", subcore_axis_name="s")


def _mm9_body(h_ref, w_ref, o_ref):
    o_ref[0] = jnp.dot(h_ref[...], w_ref[0], preferred_element_type=jnp.float32)


def _mm9(h, wcat):
    return pl.pallas_call(
        _mm9_body,
        grid=(NB, 9),
        in_specs=[
            pl.BlockSpec((BN, D), lambda j, r: (j, 0)),
            pl.BlockSpec((1, D, D), lambda j, r: (r, 0, 0)),
        ],
        out_specs=pl.BlockSpec((1, BN, D), lambda j, r: (r, j, 0)),
        out_shape=jax.ShapeDtypeStruct((9, N, D), jnp.float32),
    )(h, wcat)


def _combine_body(p_ref, t9_ref, o_ref, *, relu):
    x = p_ref[0] + p_ref[1] + t9_ref[0]
    if relu:
        x = jnp.maximum(x, 0.0)
    o_ref[...] = x


def _combine(parts, t9, relu):
    return pl.pallas_call(
        functools.partial(_combine_body, relu=relu),
        grid=(NB,),
        in_specs=[
            pl.BlockSpec((2, BN, D), lambda j: (0, j, 0)),
            pl.BlockSpec((1, BN, D), lambda j: (8, j, 0)),
        ],
        out_specs=pl.BlockSpec((BN, D), lambda j: (j, 0)),
        out_shape=jax.ShapeDtypeStruct((N, D), jnp.float32),
    )(parts, t9)


@functools.lru_cache(maxsize=None)
def _sc_scatter_kernel():
    @functools.partial(
        pl.kernel,
        out_type=jax.ShapeDtypeStruct((2, NPAD, D), jnp.float32),
        mesh=_sc_mesh(),
        scratch_types=[
            pltpu.VMEM((CH,), jnp.int32),
            pltpu.VMEM((CH,), jnp.int32),
            pltpu.VMEM((CH, D), jnp.float32),
            pltpu.VMEM_SHARED((NPAD, D), jnp.float32),
            pltpu.SemaphoreType.DMA,
        ],
    )
    def body_fn(t_hbm, gidx_hbm, dst_hbm, zeros_hbm, out_hbm,
                idx_v, dst_v, rows_v, agg_sh, sem):
        c = lax.axis_index("c")
        s = lax.axis_index("s")
        wid = s * 2 + c
        pltpu.sync_copy(zeros_hbm.at[pl.ds(s * RPT, RPT)],
                        agg_sh.at[pl.ds(s * RPT, RPT)])
        plsc.subcore_barrier()

        def body(i, carry):
            off = (wid * CPT + i) * CH
            pltpu.sync_copy(gidx_hbm.at[pl.ds(off, CH)], idx_v)
            pltpu.sync_copy(dst_hbm.at[pl.ds(off, CH)], dst_v)
            pltpu.async_copy(t_hbm.at[idx_v], rows_v, sem).wait()
            pltpu.sync_copy(rows_v, agg_sh.at[dst_v], add=True)
            return carry

        lax.fori_loop(0, CPT, body, 0)
        plsc.subcore_barrier()
        pltpu.sync_copy(agg_sh.at[pl.ds(s * RPT, RPT)],
                        out_hbm.at[c, pl.ds(s * RPT, RPT)])

    return body_fn


def _sc_scatter(t_flat, gidx_p, dst_p, zeros_rows):
    return _sc_scatter_kernel()(t_flat, gidx_p, dst_p, zeros_rows)


QPT = (Q * A) // NW


@functools.lru_cache(maxsize=None)
def _sc_gather_kernel():
    @functools.partial(
        pl.kernel,
        out_type=jax.ShapeDtypeStruct((Q * A, D), jnp.float32),
        mesh=_sc_mesh(),
        scratch_types=[
            pltpu.VMEM((QPT,), jnp.int32),
            pltpu.VMEM((QPT, D), jnp.float32),
            pltpu.SemaphoreType.DMA,
        ],
    )
    def body_fn(h_hbm, qidx_hbm, out_hbm, idx_v, rows_v, sem):
        wid = lax.axis_index("s") * 2 + lax.axis_index("c")
        base = wid * QPT
        pltpu.sync_copy(qidx_hbm.at[pl.ds(base, QPT)], idx_v)
        pltpu.async_copy(h_hbm.at[idx_v], rows_v, sem).wait()
        pltpu.sync_copy(rows_v, out_hbm.at[pl.ds(base, QPT)])

    return body_fn


def _sc_gather(h, qidx):
    return _sc_gather_kernel()(h, qidx)


def _mpqe_body(rows_ref, vid_ref, qg_ref, vemb_ref, wmsg_ref, wself_ref, o_ref):
    oh_v = (lax.broadcasted_iota(jnp.int32, (Q, V), 1) == vid_ref[...]
            ).astype(jnp.float32)
    var_part = jnp.dot(oh_v, vemb_ref[...], preferred_element_type=jnp.float32)
    xs = [rows_ref[pl.ds(a * Q, Q), :] + var_part for a in range(A)]
    for l in range(L):
        wm = wmsg_ref[l]
        ws = wself_ref[l]
        m = jnp.dot((xs[0] + xs[1] + xs[2]) * (1.0 / A), wm,
                    preferred_element_type=jnp.float32)
        xs = [jnp.maximum(jnp.dot(x, ws, preferred_element_type=jnp.float32)
                          + m, 0.0) for x in xs]
    nr = xs[0] + xs[1] + xs[2]
    oh_g = (lax.broadcasted_iota(jnp.int32, (G, Q), 0) == qg_ref[...]
            ).astype(jnp.float32)
    o_ref[...] = jnp.dot(oh_g, nr, preferred_element_type=jnp.float32)


def _mpqe(rows, vid, qg, vemb, wmsg, wself):
    return pl.pallas_call(
        _mpqe_body,
        out_shape=jax.ShapeDtypeStruct((G, D), jnp.float32),
    )(rows, vid, qg, vemb, wmsg, wself)


def kernel(edge_index, edge_type, queries, var_ids, q_graphs, emb, W_rgcn,
           W_self, var_emb, W_q_msg, W_q_self):
    src = edge_index[0].astype(jnp.int32)
    dst = edge_index[1].astype(jnp.int32)
    et = edge_type.astype(jnp.int32)
    gidx = et * N + src
    npad_e = EPAD - E
    ar = jnp.arange(npad_e, dtype=jnp.int32)
    gidx_p = jnp.concatenate([gidx, (ar * 67) % (R * N)])
    dst_p = jnp.concatenate([dst, N + (ar % (NPAD - N))])
    zeros_rows = jnp.zeros((NPAD, D), jnp.float32)
    wcat = jnp.concatenate([W_rgcn, W_self[:, None]], axis=1)

    t9 = _mm9(emb, wcat[0])
    for l in range(L):
        parts = _sc_scatter(t9.reshape(9 * N, D), gidx_p, dst_p, zeros_rows)
        if l < L - 1:
            h = _combine(parts, t9, relu=True)
            t9 = _mm9(h, wcat[l + 1])
    h = _combine(parts, t9, relu=False)

    qidx = queries.T.astype(jnp.int32).reshape(Q * A)
    rows = _sc_gather(h, qidx)
    return _mpqe(rows, var_ids.astype(jnp.int32).reshape(Q, 1),
                 q_graphs.astype(jnp.int32).reshape(1, Q), var_emb,
                 W_q_msg, W_q_self)


# speedup vs baseline: 37.6125x; 1.8220x over previous
import functools

import jax
import jax.numpy as jnp
from jax import lax
from jax.experimental import pallas as pl
from jax.experimental.pallas import tpu as pltpu
from jax.experimental.pallas import tpu_sc as plsc

N = 10000
R = 8
D = 128
L = 3
E = 320000
Q = 1024
A = 3
V = 4
G = 256

NW = 32
CH = 128
CPT = 80
EPAD = NW * CPT * CH
NPAD = N + 112
RPT = NPAD // 16

BN = 2000
NB = N // BN

@functools.lru_cache(maxsize=None)
def _sc_mesh():
    return plsc.VectorSubcoreMesh(core_axis_name="c", subcore_axis_name="s")


def _mm9_body(h_ref, w_ref, o_ref):
    o_ref[0] = jnp.dot(h_ref[...], w_ref[0], preferred_element_type=jnp.float32)


def _mm9(h, wcat):
    return pl.pallas_call(
        _mm9_body,
        grid=(NB, 9),
        in_specs=[
            pl.BlockSpec((BN, D), lambda j, r: (j, 0)),
            pl.BlockSpec((1, D, D), lambda j, r: (r, 0, 0)),
        ],
        out_specs=pl.BlockSpec((1, BN, D), lambda j, r: (r, j, 0)),
        out_shape=jax.ShapeDtypeStruct((9, N, D), jnp.float32),
    )(h, wcat)


def _combine_body(p_ref, t9_ref, o_ref, *, relu):
    x = p_ref[0] + p_ref[1] + t9_ref[0]
    if relu:
        x = jnp.maximum(x, 0.0)
    o_ref[...] = x


def _combine(parts, t9, relu):
    return pl.pallas_call(
        functools.partial(_combine_body, relu=relu),
        grid=(NB,),
        in_specs=[
            pl.BlockSpec((2, BN, D), lambda j: (0, j, 0)),
            pl.BlockSpec((1, BN, D), lambda j: (8, j, 0)),
        ],
        out_specs=pl.BlockSpec((BN, D), lambda j: (j, 0)),
        out_shape=jax.ShapeDtypeStruct((N, D), jnp.float32),
    )(parts, t9)


@functools.lru_cache(maxsize=None)
def _sc_scatter_kernel():
    @functools.partial(
        pl.kernel,
        out_type=jax.ShapeDtypeStruct((2, NPAD, D), jnp.float32),
        mesh=_sc_mesh(),
        scratch_types=[
            pltpu.VMEM((CPT, CH), jnp.int32),
            pltpu.VMEM((CH,), jnp.int32),
            pltpu.VMEM((CH,), jnp.int32),
            pltpu.VMEM((CH,), jnp.int32),
            pltpu.VMEM((CH,), jnp.int32),
            pltpu.VMEM((CH, D), jnp.float32),
            pltpu.VMEM((CH, D), jnp.float32),
            pltpu.VMEM_SHARED((NPAD, D), jnp.float32),
            pltpu.SemaphoreType.DMA,
            pltpu.SemaphoreType.DMA,
        ],
    )
    def body_fn(t_hbm, pidx_hbm, zeros_hbm, out_hbm,
                pk_v, gi0, gi1, di0, di1, rows0, rows1, agg_sh, sem0, sem1):
        c = lax.axis_index("c")
        s = lax.axis_index("s")
        wid = s * 2 + c
        gis = (gi0, gi1)
        dis = (di0, di1)
        rows = (rows0, rows1)
        sems = (sem0, sem1)

        def unpack(i, slot):
            for k in range(CH // 16):
                v = pk_v[i, pl.ds(k * 16, 16)]
                gis[slot][pl.ds(k * 16, 16)] = lax.shift_right_logical(v, 14)
                dis[slot][pl.ds(k * 16, 16)] = lax.bitwise_and(v, 0x3FFF)

        pltpu.sync_copy(pidx_hbm.at[pl.ds(wid * CPT, CPT)], pk_v)
        pltpu.sync_copy(zeros_hbm.at[pl.ds(s * RPT, RPT)],
                        agg_sh.at[pl.ds(s * RPT, RPT)])
        plsc.subcore_barrier()

        unpack(0, 0)
        pltpu.async_copy(t_hbm.at[gi0], rows0, sem0)
        unpack(1, 1)
        pltpu.async_copy(t_hbm.at[gi1], rows1, sem1)

        def half(i, slot):
            pltpu.make_async_copy(t_hbm.at[gis[slot]], rows[slot],
                                  sems[slot]).wait()
            pltpu.sync_copy(rows[slot], agg_sh.at[dis[slot]], add=True)
            unpack(i + 2, slot)
            pltpu.async_copy(t_hbm.at[gis[slot]], rows[slot], sems[slot])

        def body(j, carry):
            half(2 * j, 0)
            half(2 * j + 1, 1)
            return carry

        lax.fori_loop(0, CPT // 2 - 1, body, 0)
        pltpu.make_async_copy(t_hbm.at[gi0], rows0, sem0).wait()
        pltpu.sync_copy(rows0, agg_sh.at[di0], add=True)
        pltpu.make_async_copy(t_hbm.at[gi1], rows1, sem1).wait()
        pltpu.sync_copy(rows1, agg_sh.at[di1], add=True)

        plsc.subcore_barrier()
        pltpu.sync_copy(agg_sh.at[pl.ds(s * RPT, RPT)],
                        out_hbm.at[c, pl.ds(s * RPT, RPT)])

    return body_fn


def _sc_scatter(t_flat, pidx, zeros_rows):
    return _sc_scatter_kernel()(t_flat, pidx.reshape(NW * CPT, CH), zeros_rows)


QPT = (Q * A) // NW


@functools.lru_cache(maxsize=None)
def _sc_gather_kernel():
    @functools.partial(
        pl.kernel,
        out_type=jax.ShapeDtypeStruct((Q * A, D), jnp.float32),
        mesh=_sc_mesh(),
        scratch_types=[
            pltpu.VMEM((QPT,), jnp.int32),
            pltpu.VMEM((QPT, D), jnp.float32),
            pltpu.SemaphoreType.DMA,
        ],
    )
    def body_fn(h_hbm, qidx_hbm, out_hbm, idx_v, rows_v, sem):
        wid = lax.axis_index("s") * 2 + lax.axis_index("c")
        base = wid * QPT
        pltpu.sync_copy(qidx_hbm.at[pl.ds(base, QPT)], idx_v)
        pltpu.async_copy(h_hbm.at[idx_v], rows_v, sem).wait()
        pltpu.sync_copy(rows_v, out_hbm.at[pl.ds(base, QPT)])

    return body_fn


def _sc_gather(h, qidx):
    return _sc_gather_kernel()(h, qidx)


def _mpqe_body(rows_ref, vid_ref, qg_ref, vemb_ref, wmsg_ref, wself_ref, o_ref):
    oh_v = (lax.broadcasted_iota(jnp.int32, (Q, V), 1) == vid_ref[...]
            ).astype(jnp.float32)
    var_part = jnp.dot(oh_v, vemb_ref[...], preferred_element_type=jnp.float32)
    xs = [rows_ref[pl.ds(a * Q, Q), :] + var_part for a in range(A)]
    for l in range(L):
        wm = wmsg_ref[l]
        ws = wself_ref[l]
        m = jnp.dot((xs[0] + xs[1] + xs[2]) * (1.0 / A), wm,
                    preferred_element_type=jnp.float32)
        xs = [jnp.maximum(jnp.dot(x, ws, preferred_element_type=jnp.float32)
                          + m, 0.0) for x in xs]
    nr = xs[0] + xs[1] + xs[2]
    oh_g = (lax.broadcasted_iota(jnp.int32, (G, Q), 0) == qg_ref[...]
            ).astype(jnp.float32)
    o_ref[...] = jnp.dot(oh_g, nr, preferred_element_type=jnp.float32)


def _mpqe(rows, vid, qg, vemb, wmsg, wself):
    return pl.pallas_call(
        _mpqe_body,
        out_shape=jax.ShapeDtypeStruct((G, D), jnp.float32),
    )(rows, vid, qg, vemb, wmsg, wself)


def kernel(edge_index, edge_type, queries, var_ids, q_graphs, emb, W_rgcn,
           W_self, var_emb, W_q_msg, W_q_self):
    src = edge_index[0].astype(jnp.int32)
    dst = edge_index[1].astype(jnp.int32)
    et = edge_type.astype(jnp.int32)
    gidx = et * N + src
    npad_e = EPAD - E
    ar = jnp.arange(npad_e, dtype=jnp.int32)
    gidx_p = jnp.concatenate([gidx, (ar * 67) % (R * N)])
    dst_p = jnp.concatenate([dst, N + (ar % (NPAD - N))])
    pidx = gidx_p * 16384 + dst_p
    zeros_rows = jnp.zeros((NPAD, D), jnp.float32)
    wcat = jnp.concatenate([W_rgcn, W_self[:, None]], axis=1)

    t9 = _mm9(emb, wcat[0])
    for l in range(L):
        parts = _sc_scatter(t9.reshape(9 * N, D), pidx, zeros_rows)
        if l < L - 1:
            h = _combine(parts, t9, relu=True)
            t9 = _mm9(h, wcat[l + 1])
    h = _combine(parts, t9, relu=False)

    qidx = queries.T.astype(jnp.int32).reshape(Q * A)
    rows = _sc_gather(h, qidx)
    return _mpqe(rows, var_ids.astype(jnp.int32).reshape(Q, 1),
                 q_graphs.astype(jnp.int32).reshape(1, Q), var_emb,
                 W_q_msg, W_q_self)


# speedup vs baseline: 38.2471x; 1.0169x over previous
import functools

import jax
import jax.numpy as jnp
from jax import lax
from jax.experimental import pallas as pl
from jax.experimental.pallas import tpu as pltpu
from jax.experimental.pallas import tpu_sc as plsc

N = 10000
R = 8
D = 128
L = 3
E = 320000
Q = 1024
A = 3
V = 4
G = 256

NW = 32
CH = 128
CPT = 80
EPAD = NW * CPT * CH
NPAD = N + 112
RPT = NPAD // 16

BN = 2000
NB = N // BN

@functools.lru_cache(maxsize=None)
def _sc_mesh():
    return plsc.VectorSubcoreMesh(core_axis_name="c", subcore_axis_name="s")


def _mm9_body(h_ref, w_ref, o_ref):
    o_ref[0] = jnp.dot(h_ref[...], w_ref[0], preferred_element_type=jnp.float32)


def _mm9(h, wcat):
    return pl.pallas_call(
        _mm9_body,
        grid=(NB, 9),
        in_specs=[
            pl.BlockSpec((BN, D), lambda j, r: (j, 0)),
            pl.BlockSpec((1, D, D), lambda j, r: (r, 0, 0)),
        ],
        out_specs=pl.BlockSpec((1, BN, D), lambda j, r: (r, j, 0)),
        out_shape=jax.ShapeDtypeStruct((9, N, D), jnp.float32),
    )(h, wcat)


def _mm9f_body(p_ref, t9p_ref, w_ref, o_ref, h_ref):
    @pl.when(pl.program_id(1) == 0)
    def _():
        h_ref[...] = jnp.maximum(p_ref[0] + p_ref[1] + t9p_ref[0], 0.0)

    o_ref[0] = jnp.dot(h_ref[...], w_ref[0], preferred_element_type=jnp.float32)


def _mm9_fused(parts, t9p, wcat):
    return pl.pallas_call(
        _mm9f_body,
        grid=(NB, 9),
        in_specs=[
            pl.BlockSpec((2, BN, D), lambda j, r: (0, j, 0)),
            pl.BlockSpec((1, BN, D), lambda j, r: (8, j, 0)),
            pl.BlockSpec((1, D, D), lambda j, r: (r, 0, 0)),
        ],
        out_specs=pl.BlockSpec((1, BN, D), lambda j, r: (r, j, 0)),
        out_shape=jax.ShapeDtypeStruct((9, N, D), jnp.float32),
        scratch_shapes=[pltpu.VMEM((BN, D), jnp.float32)],
    )(parts, t9p, wcat)


def _combine_body(p_ref, t9_ref, o_ref, *, relu):
    x = p_ref[0] + p_ref[1] + t9_ref[0]
    if relu:
        x = jnp.maximum(x, 0.0)
    o_ref[...] = x


def _combine(parts, t9, relu):
    return pl.pallas_call(
        functools.partial(_combine_body, relu=relu),
        grid=(NB,),
        in_specs=[
            pl.BlockSpec((2, BN, D), lambda j: (0, j, 0)),
            pl.BlockSpec((1, BN, D), lambda j: (8, j, 0)),
        ],
        out_specs=pl.BlockSpec((BN, D), lambda j: (j, 0)),
        out_shape=jax.ShapeDtypeStruct((N, D), jnp.float32),
    )(parts, t9)


@functools.lru_cache(maxsize=None)
def _sc_scatter_kernel():
    @functools.partial(
        pl.kernel,
        out_type=jax.ShapeDtypeStruct((2, NPAD, D), jnp.float32),
        mesh=_sc_mesh(),
        scratch_types=[
            pltpu.VMEM((CPT, CH), jnp.int32),
            pltpu.VMEM((CH,), jnp.int32),
            pltpu.VMEM((CH,), jnp.int32),
            pltpu.VMEM((CH,), jnp.int32),
            pltpu.VMEM((CH,), jnp.int32),
            pltpu.VMEM((CH, D), jnp.float32),
            pltpu.VMEM((CH, D), jnp.float32),
            pltpu.VMEM_SHARED((NPAD, D), jnp.float32),
            pltpu.SemaphoreType.DMA,
            pltpu.SemaphoreType.DMA,
        ],
    )
    def body_fn(t_hbm, pidx_hbm, zeros_hbm, out_hbm,
                pk_v, gi0, gi1, di0, di1, rows0, rows1, agg_sh, sem0, sem1):
        c = lax.axis_index("c")
        s = lax.axis_index("s")
        wid = s * 2 + c
        gis = (gi0, gi1)
        dis = (di0, di1)
        rows = (rows0, rows1)
        sems = (sem0, sem1)

        def unpack(i, slot):
            for k in range(CH // 16):
                v = pk_v[i, pl.ds(k * 16, 16)]
                gis[slot][pl.ds(k * 16, 16)] = lax.shift_right_logical(v, 14)
                dis[slot][pl.ds(k * 16, 16)] = lax.bitwise_and(v, 0x3FFF)

        pltpu.sync_copy(pidx_hbm.at[pl.ds(wid * CPT, CPT)], pk_v)
        pltpu.sync_copy(zeros_hbm.at[pl.ds(s * RPT, RPT)],
                        agg_sh.at[pl.ds(s * RPT, RPT)])
        plsc.subcore_barrier()

        unpack(0, 0)
        pltpu.async_copy(t_hbm.at[gi0], rows0, sem0)
        unpack(1, 1)
        pltpu.async_copy(t_hbm.at[gi1], rows1, sem1)

        def half(i, slot):
            pltpu.make_async_copy(t_hbm.at[gis[slot]], rows[slot],
                                  sems[slot]).wait()
            pltpu.sync_copy(rows[slot], agg_sh.at[dis[slot]], add=True)
            unpack(i + 2, slot)
            pltpu.async_copy(t_hbm.at[gis[slot]], rows[slot], sems[slot])

        def body(j, carry):
            half(2 * j, 0)
            half(2 * j + 1, 1)
            return carry

        lax.fori_loop(0, CPT // 2 - 1, body, 0)
        pltpu.make_async_copy(t_hbm.at[gi0], rows0, sem0).wait()
        pltpu.sync_copy(rows0, agg_sh.at[di0], add=True)
        pltpu.make_async_copy(t_hbm.at[gi1], rows1, sem1).wait()
        pltpu.sync_copy(rows1, agg_sh.at[di1], add=True)

        plsc.subcore_barrier()
        pltpu.sync_copy(agg_sh.at[pl.ds(s * RPT, RPT)],
                        out_hbm.at[c, pl.ds(s * RPT, RPT)])

    return body_fn


def _sc_scatter(t_flat, pidx, zeros_rows):
    return _sc_scatter_kernel()(t_flat, pidx.reshape(NW * CPT, CH), zeros_rows)


QPT = (Q * A) // NW


@functools.lru_cache(maxsize=None)
def _sc_gather_kernel():
    @functools.partial(
        pl.kernel,
        out_type=jax.ShapeDtypeStruct((Q * A, D), jnp.float32),
        mesh=_sc_mesh(),
        scratch_types=[
            pltpu.VMEM((QPT,), jnp.int32),
            pltpu.VMEM((QPT, D), jnp.float32),
            pltpu.SemaphoreType.DMA,
        ],
    )
    def body_fn(h_hbm, qidx_hbm, out_hbm, idx_v, rows_v, sem):
        wid = lax.axis_index("s") * 2 + lax.axis_index("c")
        base = wid * QPT
        pltpu.sync_copy(qidx_hbm.at[pl.ds(base, QPT)], idx_v)
        pltpu.async_copy(h_hbm.at[idx_v], rows_v, sem).wait()
        pltpu.sync_copy(rows_v, out_hbm.at[pl.ds(base, QPT)])

    return body_fn


def _sc_gather(h, qidx):
    return _sc_gather_kernel()(h, qidx)


def _mpqe_body(rows_ref, vid_ref, qg_ref, vemb_ref, wmsg_ref, wself_ref, o_ref):
    oh_v = (lax.broadcasted_iota(jnp.int32, (Q, V), 1) == vid_ref[...]
            ).astype(jnp.float32)
    var_part = jnp.dot(oh_v, vemb_ref[...], preferred_element_type=jnp.float32)
    xs = [rows_ref[pl.ds(a * Q, Q), :] + var_part for a in range(A)]
    for l in range(L):
        wm = wmsg_ref[l]
        ws = wself_ref[l]
        m = jnp.dot((xs[0] + xs[1] + xs[2]) * (1.0 / A), wm,
                    preferred_element_type=jnp.float32)
        xs = [jnp.maximum(jnp.dot(x, ws, preferred_element_type=jnp.float32)
                          + m, 0.0) for x in xs]
    nr = xs[0] + xs[1] + xs[2]
    oh_g = (lax.broadcasted_iota(jnp.int32, (G, Q), 0) == qg_ref[...]
            ).astype(jnp.float32)
    o_ref[...] = jnp.dot(oh_g, nr, preferred_element_type=jnp.float32)


def _mpqe(rows, vid, qg, vemb, wmsg, wself):
    return pl.pallas_call(
        _mpqe_body,
        out_shape=jax.ShapeDtypeStruct((G, D), jnp.float32),
    )(rows, vid, qg, vemb, wmsg, wself)


def kernel(edge_index, edge_type, queries, var_ids, q_graphs, emb, W_rgcn,
           W_self, var_emb, W_q_msg, W_q_self):
    src = edge_index[0].astype(jnp.int32)
    dst = edge_index[1].astype(jnp.int32)
    et = edge_type.astype(jnp.int32)
    gidx = et * N + src
    npad_e = EPAD - E
    ar = jnp.arange(npad_e, dtype=jnp.int32)
    gidx_p = jnp.concatenate([gidx, (ar * 67) % (R * N)])
    dst_p = jnp.concatenate([dst, N + (ar % (NPAD - N))])
    pidx = gidx_p * 16384 + dst_p
    zeros_rows = jnp.zeros((NPAD, D), jnp.float32)
    wcat = jnp.concatenate([W_rgcn, W_self[:, None]], axis=1)

    t9 = _mm9(emb, wcat[0])
    for l in range(L):
        parts = _sc_scatter(t9.reshape(9 * N, D), pidx, zeros_rows)
        if l < L - 1:
            t9 = _mm9_fused(parts, t9, wcat[l + 1])
    h = _combine(parts, t9, relu=False)

    qidx = queries.T.astype(jnp.int32).reshape(Q * A)
    rows = _sc_gather(h, qidx)
    return _mpqe(rows, var_ids.astype(jnp.int32).reshape(Q, 1),
                 q_graphs.astype(jnp.int32).reshape(1, Q), var_emb,
                 W_q_msg, W_q_self)
